# baseline (speedup 1.0000x reference)
import functools

import jax
import jax.numpy as jnp
from jax import lax
from jax.experimental import pallas as pl
from jax.experimental.pallas import tpu as pltpu
from jax.experimental.pallas import tpu_sc as plsc

NE = 1024
D = 64
N = 18432

RB = 6144
GRID = N // RB

NW = 32
BPW = N // NW
NCH, CH = 6, 96


CW = 256


def _argmin_body(zsq_ref, esq_ref, z_ref, emb_ref, idx_ref):
    z = z_ref[...]
    zsq = zsq_ref[...]
    run_min = run_idx = None
    for c in range(NE // CW):
        m = lax.dot_general(
            z, emb_ref[pl.ds(c * CW, CW), :],
            (((1,), (1,)), ((), ())),
            preferred_element_type=jnp.float32,
            precision=lax.Precision.DEFAULT,
        )
        d = zsq - 2.0 * m + esq_ref[:, pl.ds(c * CW, CW)]
        jc = lax.broadcasted_iota(jnp.int32, d.shape, 1) + jnp.int32(c * CW)
        if c == 0:
            run_min, run_idx = d, jc
        else:
            pred = d < run_min
            run_min = jnp.where(pred, d, run_min)
            run_idx = jnp.where(pred, jc, run_idx)
    mn = jnp.min(run_min, axis=1, keepdims=True)
    cand = jnp.where(run_min == mn, run_idx, jnp.int32(NE))
    idx_ref[...] = jnp.min(cand, axis=1)


_nearest = pl.pallas_call(
    _argmin_body,
    grid=(GRID,),
    in_specs=[
        pl.BlockSpec((RB, 1), lambda i: (i, 0)),
        pl.BlockSpec((1, NE), lambda i: (0, 0)),
        pl.BlockSpec((RB, D), lambda i: (i, 0)),
        pl.BlockSpec((NE, D), lambda i: (0, 0)),
    ],
    out_specs=pl.BlockSpec((RB,), lambda i: (i,)),
    out_shape=jax.ShapeDtypeStruct((N,), jnp.int32),
)


@functools.partial(
    pl.kernel,
    mesh=plsc.VectorSubcoreMesh(core_axis_name="c", subcore_axis_name="s"),
    out_type=jax.ShapeDtypeStruct((NW, BPW, D), jnp.float32),
    scratch_types=[
        pltpu.VMEM((NCH, CH), jnp.int32),
        pltpu.VMEM((2, CH, 2 * D), jnp.float32),
        pltpu.VMEM((BPW, D), jnp.float32),
        pltpu.SemaphoreType.DMA,
        pltpu.SemaphoreType.DMA,
    ],
)
def _gather(emb_hbm, idx_hbm, out_hbm, idx_v, rows_v, packed_v, sem0, sem1):
    wid = lax.axis_index("s") * 2 + lax.axis_index("c")
    sems = (sem0, sem1)
    pltpu.sync_copy(idx_hbm.at[wid], idx_v)

    def _fire(j):
        return pltpu.async_copy(
            emb_hbm.at[idx_v.at[j]], rows_v.at[j % 2], sems[j % 2]
        )

    pend = _fire(0)
    for j in range(NCH):
        pend.wait()
        if j + 1 < NCH:
            pend = _fire(j + 1)
        buf = j % 2

        def _compact(r, carry):
            for k in range(D // 16):
                packed_v[j * CH + r, pl.ds(k * 16, 16)] = rows_v[
                    buf, r, pl.ds(k * 16, 16)
                ]
            return carry

        lax.fori_loop(0, CH, _compact, 0)
    pltpu.sync_copy(packed_v, out_hbm.at[wid])


def kernel(z_e, embeddings):
    z_flat = z_e.reshape(-1, D)
    zsq = jnp.sum(jnp.square(z_flat), axis=1, keepdims=True)
    esq = jnp.sum(jnp.square(embeddings), axis=1)[None, :]
    idx_flat = _nearest(zsq, esq, z_flat, embeddings)
    z = idx_flat.reshape(z_e.shape[:-1])
    emb2 = jnp.concatenate([embeddings, embeddings], axis=1)
    z_q = _gather(emb2, idx_flat.reshape(NW, NCH, CH)).reshape(z_e.shape)
    return z, z_q


# speedup vs baseline: 1.0859x; 1.0859x over previous
import functools

import jax
import jax.numpy as jnp
from jax import lax
from jax.experimental import pallas as pl
from jax.experimental.pallas import tpu as pltpu
from jax.experimental.pallas import tpu_sc as plsc

NE = 1024
D = 64
B = 32
S = 576
N = B * S

RB = 4608
GRID = N // RB

CW = 256

NW = 32
BPW = N // NW
NCH, CH = 6, 96


def _argmin_body(zsq_ref, esq_ref, zt_ref, emb_ref, idx_ref):
    zt = zt_ref[...]
    zsq = zsq_ref[...]
    run_min = run_idx = None
    for c in range(NE // CW):
        m = lax.dot_general(
            emb_ref[pl.ds(c * CW, CW), :], zt,
            (((1,), (0,)), ((), ())),
            preferred_element_type=jnp.float32,
            precision=lax.Precision.DEFAULT,
        )
        d = zsq - 2.0 * m + esq_ref[pl.ds(c * CW, CW), :]
        jc = lax.broadcasted_iota(jnp.int32, d.shape, 0) + jnp.int32(c * CW)
        if c == 0:
            run_min, run_idx = d, jc
        else:
            pred = d < run_min
            run_min = jnp.where(pred, d, run_min)
            run_idx = jnp.where(pred, jc, run_idx)
    mn = jnp.min(run_min, axis=0, keepdims=True)
    cand = jnp.where(run_min == mn, run_idx, jnp.int32(NE))
    idx_ref[...] = jnp.min(cand, axis=0).reshape(1, RB)


_nearest = pl.pallas_call(
    _argmin_body,
    grid=(GRID,),
    in_specs=[
        pl.BlockSpec((1, RB), lambda i: (0, i)),
        pl.BlockSpec((NE, 1), lambda i: (0, 0)),
        pl.BlockSpec((D, RB), lambda i: (0, i)),
        pl.BlockSpec((NE, D), lambda i: (0, 0)),
    ],
    out_specs=pl.BlockSpec((1, RB), lambda i: (0, i)),
    out_shape=jax.ShapeDtypeStruct((1, N), jnp.int32),
)


@functools.partial(
    pl.kernel,
    mesh=plsc.VectorSubcoreMesh(core_axis_name="c", subcore_axis_name="s"),
    out_type=jax.ShapeDtypeStruct((NW, BPW, D), jnp.float32),
    scratch_types=[
        pltpu.VMEM((BPW,), jnp.int32),
        pltpu.VMEM((2, CH, 2 * D), jnp.float32),
        pltpu.VMEM((BPW, D), jnp.float32),
        pltpu.SemaphoreType.DMA,
        pltpu.SemaphoreType.DMA,
    ],
)
def _gather(emb_hbm, idx_hbm, out_hbm, idx_v, rows_v, packed_v, sem0, sem1):
    wid = lax.axis_index("s") * 2 + lax.axis_index("c")
    sems = (sem0, sem1)
    pltpu.sync_copy(idx_hbm.at[wid], idx_v)

    def _fire(j):
        return pltpu.async_copy(
            emb_hbm.at[idx_v.at[pl.ds(j * CH, CH)]], rows_v.at[j % 2],
            sems[j % 2],
        )

    pend = _fire(0)
    for j in range(NCH):
        pend.wait()
        if j + 1 < NCH:
            pend = _fire(j + 1)
        buf = j % 2

        def _compact(r, carry):
            for k in range(D // 16):
                packed_v[j * CH + r, pl.ds(k * 16, 16)] = rows_v[
                    buf, r, pl.ds(k * 16, 16)
                ]
            return carry

        lax.fori_loop(0, CH, _compact, 0)
    pltpu.sync_copy(packed_v, out_hbm.at[wid])


def kernel(z_e, embeddings):
    z_flat = z_e.reshape(-1, D)
    zsq = jnp.sum(jnp.square(z_flat), axis=1, keepdims=True).reshape(1, N)
    esq = jnp.sum(jnp.square(embeddings), axis=1)[:, None]
    z_t = z_flat.T
    idx_row = _nearest(zsq, esq, z_t, embeddings)
    z = idx_row.reshape(B, S)
    emb2 = jnp.concatenate([embeddings, embeddings], axis=1)
    z_q = _gather(emb2, z).reshape(z_e.shape)
    return z, z_q


# speedup vs baseline: 1.1383x; 1.0483x over previous
import functools

import jax
import jax.numpy as jnp
from jax import lax
from jax.experimental import pallas as pl
from jax.experimental.pallas import tpu as pltpu
from jax.experimental.pallas import tpu_sc as plsc

NE = 1024
D = 64
B = 32
S = 576
N = B * S

RB = 4608
GRID = N // RB

CW = 256

NW = 32
BPW = N // NW
NCH, CH = 6, 96


def _argmin_body(zsq_ref, esq_ref, z_ref, emb_ref, idx_ref):
    z = z_ref[...].reshape(RB, D)
    zsq = zsq_ref[...]
    run_min = run_idx = None
    for c in range(NE // CW):
        m = lax.dot_general(
            emb_ref[pl.ds(c * CW, CW), :], z,
            (((1,), (1,)), ((), ())),
            preferred_element_type=jnp.float32,
            precision=lax.Precision.DEFAULT,
        )
        d = zsq - 2.0 * m + esq_ref[pl.ds(c * CW, CW), :]
        jc = lax.broadcasted_iota(jnp.int32, d.shape, 0) + jnp.int32(c * CW)
        if c == 0:
            run_min, run_idx = d, jc
        else:
            pred = d < run_min
            run_min = jnp.where(pred, d, run_min)
            run_idx = jnp.where(pred, jc, run_idx)
    mn = jnp.min(run_min, axis=0, keepdims=True)
    cand = jnp.where(run_min == mn, run_idx, jnp.int32(NE))
    idx_ref[...] = jnp.min(cand, axis=0).reshape(1, RB)


_nearest = pl.pallas_call(
    _argmin_body,
    grid=(GRID,),
    in_specs=[
        pl.BlockSpec((1, RB), lambda i: (0, i)),
        pl.BlockSpec((NE, 1), lambda i: (0, 0)),
        pl.BlockSpec((8, S, D), lambda i: (i, 0, 0)),
        pl.BlockSpec((NE, D), lambda i: (0, 0)),
    ],
    out_specs=pl.BlockSpec((1, RB), lambda i: (0, i)),
    out_shape=jax.ShapeDtypeStruct((1, N), jnp.int32),
)


@functools.partial(
    pl.kernel,
    mesh=plsc.VectorSubcoreMesh(core_axis_name="c", subcore_axis_name="s"),
    out_type=jax.ShapeDtypeStruct((NW, BPW, D), jnp.float32),
    scratch_types=[
        pltpu.VMEM((BPW,), jnp.int32),
        pltpu.VMEM((2, CH, 2 * D), jnp.float32),
        pltpu.VMEM((BPW, D), jnp.float32),
        pltpu.SemaphoreType.DMA,
        pltpu.SemaphoreType.DMA,
    ],
)
def _gather(emb_hbm, idx_hbm, out_hbm, idx_v, rows_v, packed_v, sem0, sem1):
    wid = lax.axis_index("s") * 2 + lax.axis_index("c")
    sems = (sem0, sem1)
    pltpu.sync_copy(idx_hbm.at[wid], idx_v)

    def _fire(j):
        return pltpu.async_copy(
            emb_hbm.at[idx_v.at[pl.ds(j * CH, CH)]], rows_v.at[j % 2],
            sems[j % 2],
        )

    pend = _fire(0)
    for j in range(NCH):
        pend.wait()
        if j + 1 < NCH:
            pend = _fire(j + 1)
        buf = j % 2

        def _compact(r, carry):
            for k in range(D // 16):
                packed_v[j * CH + r, pl.ds(k * 16, 16)] = rows_v[
                    buf, r, pl.ds(k * 16, 16)
                ]
            return carry

        lax.fori_loop(0, CH, _compact, 0)
    pltpu.sync_copy(packed_v, out_hbm.at[wid])


def kernel(z_e, embeddings):
    z_flat = z_e.reshape(-1, D)
    zsq = jnp.sum(jnp.square(z_flat), axis=1, keepdims=True).reshape(1, N)
    esq = jnp.sum(jnp.square(embeddings), axis=1)[:, None]
    idx_row = _nearest(zsq, esq, z_e, embeddings)
    z = idx_row.reshape(B, S)
    emb2 = jnp.concatenate([embeddings, embeddings], axis=1)
    z_q = _gather(emb2, z).reshape(z_e.shape)
    return z, z_q


# speedup vs baseline: 1.1698x; 1.0277x over previous
import functools

import jax
import jax.numpy as jnp
from jax import lax
from jax.experimental import pallas as pl
from jax.experimental.pallas import tpu as pltpu
from jax.experimental.pallas import tpu_sc as plsc

NE = 1024
D = 64
B = 32
S = 576
N = B * S

RB = 4608
GRID = N // RB

CW = 128

NW = 32
BPW = N // NW
NCH, CH = 6, 96


def _argmin_body(zsq_ref, esq_ref, z_ref, emb_ref, idx_ref):
    z = z_ref[...].reshape(RB, D)
    zsq = zsq_ref[...]
    run_min = run_idx = None
    for c in range(NE // CW):
        m = lax.dot_general(
            emb_ref[pl.ds(c * CW, CW), :], z,
            (((1,), (1,)), ((), ())),
            preferred_element_type=jnp.float32,
            precision=lax.Precision.DEFAULT,
        )
        d = zsq - 2.0 * m + esq_ref[pl.ds(c * CW, CW), :]
        jc = lax.broadcasted_iota(jnp.int32, d.shape, 0) + jnp.int32(c * CW)
        if c == 0:
            run_min, run_idx = d, jc
        else:
            pred = d < run_min
            run_min = jnp.where(pred, d, run_min)
            run_idx = jnp.where(pred, jc, run_idx)
    mn = jnp.min(run_min, axis=0, keepdims=True)
    cand = jnp.where(run_min == mn, run_idx, jnp.int32(NE))
    idx_ref[...] = jnp.min(cand, axis=0).reshape(1, RB)


_nearest = pl.pallas_call(
    _argmin_body,
    grid=(GRID,),
    in_specs=[
        pl.BlockSpec((1, RB), lambda i: (0, i)),
        pl.BlockSpec((NE, 1), lambda i: (0, 0)),
        pl.BlockSpec((8, S, D), lambda i: (i, 0, 0)),
        pl.BlockSpec((NE, D), lambda i: (0, 0)),
    ],
    out_specs=pl.BlockSpec((1, RB), lambda i: (0, i)),
    out_shape=jax.ShapeDtypeStruct((1, N), jnp.int32),
)


@functools.partial(
    pl.kernel,
    mesh=plsc.VectorSubcoreMesh(core_axis_name="c", subcore_axis_name="s"),
    out_type=jax.ShapeDtypeStruct((NW, BPW, D), jnp.float32),
    scratch_types=[
        pltpu.VMEM((BPW,), jnp.int32),
        pltpu.VMEM((2, CH, 2 * D), jnp.float32),
        pltpu.VMEM((BPW, D), jnp.float32),
        pltpu.SemaphoreType.DMA,
        pltpu.SemaphoreType.DMA,
    ],
)
def _gather(emb_hbm, idx_hbm, out_hbm, idx_v, rows_v, packed_v, sem0, sem1):
    wid = lax.axis_index("s") * 2 + lax.axis_index("c")
    sems = (sem0, sem1)
    pltpu.sync_copy(idx_hbm.at[wid], idx_v)

    def _fire(j):
        return pltpu.async_copy(
            emb_hbm.at[idx_v.at[pl.ds(j * CH, CH)]], rows_v.at[j % 2],
            sems[j % 2],
        )

    pend = _fire(0)
    for j in range(NCH):
        pend.wait()
        if j + 1 < NCH:
            pend = _fire(j + 1)
        buf = j % 2

        def _compact(r, carry):
            for k in range(D // 16):
                packed_v[j * CH + r, pl.ds(k * 16, 16)] = rows_v[
                    buf, r, pl.ds(k * 16, 16)
                ]
            return carry

        lax.fori_loop(0, CH, _compact, 0)
    pltpu.sync_copy(packed_v, out_hbm.at[wid])


def kernel(z_e, embeddings):
    z_flat = z_e.reshape(-1, D)
    zsq = jnp.sum(jnp.square(z_flat), axis=1, keepdims=True).reshape(1, N)
    esq = jnp.sum(jnp.square(embeddings), axis=1)[:, None]
    idx_row = _nearest(zsq, esq, z_e, embeddings)
    z = idx_row.reshape(B, S)
    emb2 = jnp.concatenate([embeddings, embeddings], axis=1)
    z_q = _gather(emb2, z).reshape(z_e.shape)
    return z, z_q
